# baseline (speedup 1.0000x reference)
import functools

import jax
import jax.numpy as jnp
from jax import lax
from jax.experimental import pallas as pl
from jax.experimental.pallas import tpu as pltpu
from jax.experimental.pallas import tpu_sc as plsc

_NC = 2
_NS = 16
_NW = _NC * _NS


def _full(shape):
    return pl.BlockSpec(shape, lambda i: (0,) * len(shape))


def _round_bf16_bits(v):
    b = lax.bitcast_convert_type(v, jnp.uint32)
    return (b + jnp.uint32(0x7FFF) + ((b >> 16) & jnp.uint32(1))) >> 16


def _node_body(x_ref, nt_ref, nnW1, nnb1, nnW2, nnb2, gW1x, gW1t, gb1,
               clW, clb, T_ref, skip_ref):
    x = x_ref[...]
    h = jnp.maximum(x @ nnW1[...] + nnb1[...], 0.0) @ nnW2[...] + nnb2[...]
    g = x @ gW1x[...] + nt_ref[...] * gW1t[...] + gb1[...]
    w = _round_bf16_bits(h) | (_round_bf16_bits(g) << 16)
    T_ref[...] = lax.bitcast_convert_type(w, jnp.int32)
    skip_ref[...] = x @ clW[...] + clb[...]


def _edge_body(ea_ref, G_ref, enW1, enb1, enW2, enb2, msgW, msgb,
               gW1e, gW2, gb2, out_ref):
    ea = ea_ref[...]
    he = jnp.maximum(ea @ enW1[...] + enb1[...], 0.0) @ enW2[...] + enb2[...]
    w = lax.bitcast_convert_type(G_ref[...], jnp.uint32)
    gh = lax.bitcast_convert_type(w << 16, jnp.float32)
    gg = lax.bitcast_convert_type(w & jnp.uint32(0xFFFF0000), jnp.float32)
    m = (he * gh) @ msgW[...] + msgb[...]
    gt = jnp.maximum(ea @ gW1e[...] + gg, 0.0) @ gW2[...] + gb2[...]
    out_ref[...] = m * jax.nn.sigmoid(gt)


def _final_body(skip_ref, *rest):
    p_refs = rest[:-5]
    lng, lnb, outW, outb, out_ref = rest[-5:]
    o = skip_ref[...]
    for p in p_refs:
        o = o + p[...]
    mu = jnp.mean(o, axis=-1, keepdims=True)
    var = jnp.mean((o - mu) ** 2, axis=-1, keepdims=True)
    o = (o - mu) / jnp.sqrt(var + 1e-5) * lng[...] + lnb[...]
    out_ref[...] = jnp.maximum(o, 0.0) @ outW[...] + outb[...]


def _sc_gather(table, idx3):
    nt, d = table.shape
    nw, nch, ch = idx3.shape
    bpw = nch * ch
    b = nw * bpw
    nbuf = 3
    lag = 2
    ngrp = -(-(nch + lag) // nbuf)
    mesh = plsc.VectorSubcoreMesh(core_axis_name="c", subcore_axis_name="s")

    @functools.partial(
        pl.kernel,
        out_type=jax.ShapeDtypeStruct((b, d), table.dtype),
        mesh=mesh,
        scratch_types=[
            pltpu.VMEM((nch, ch), jnp.int32),
            [pltpu.VMEM((ch, d), table.dtype)] * nbuf,
            pltpu.VMEM_SHARED((nt, d), table.dtype),
            [pltpu.SemaphoreType.DMA] * nbuf,
            [pltpu.SemaphoreType.DMA] * nbuf,
        ],
    )
    def gk(table_hbm, idx_hbm, out_hbm, idx_v, bufs, tab_sh, gsems, wsems):
        cid = lax.axis_index("c")
        sid = lax.axis_index("s")
        wid = sid * _NC + cid
        base = wid * bpw
        @pl.when(sid == 0)
        def _():
            pltpu.sync_copy(table_hbm, tab_sh)

        pltpu.sync_copy(idx_hbm.at[wid], idx_v)
        plsc.subcore_barrier()

        def step(i, bb):
            bi = bb
            bj = (bb - lag) % nbuf

            @pl.when(jnp.logical_and(i >= nbuf, i < nch))
            def _():
                pltpu.make_async_copy(
                    bufs[bi], out_hbm.at[pl.ds(base, ch)], wsems[bi]).wait()

            @pl.when(i < nch)
            def _():
                pltpu.async_copy(tab_sh.at[idx_v.at[i]], bufs[bi], gsems[bi])

            j = i - lag

            @pl.when(jnp.logical_and(j >= 0, j < nch))
            def _():
                pltpu.make_async_copy(
                    tab_sh.at[idx_v.at[0]], bufs[bj], gsems[bj]).wait()
                pltpu.async_copy(
                    bufs[bj], out_hbm.at[pl.ds(base + j * ch, ch)], wsems[bj])

        def body(g, carry):
            for bb in range(nbuf):
                step(g * nbuf + bb, bb)
            return carry

        lax.fori_loop(0, ngrp, body, 0)
        for bb in range(min(nbuf, nch)):
            pltpu.make_async_copy(
                bufs[bb], out_hbm.at[pl.ds(base, ch)], wsems[bb]).wait()

    return gk(table, idx3)


def _sc_scatter(msg, idx3, zblk):
    b, d = msg.shape
    nw, nch, ch = idx3.shape
    bpw = nch * ch
    nps = zblk.shape[0]
    nseg = nps * _NS
    mesh = plsc.VectorSubcoreMesh(core_axis_name="c", subcore_axis_name="s")

    nbuf = 3
    lag = 2
    ngrp = -(-(nch + lag) // nbuf)

    @functools.partial(
        pl.kernel,
        out_type=jax.ShapeDtypeStruct((_NC, nseg, d), jnp.float32),
        mesh=mesh,
        scratch_types=[
            pltpu.VMEM((nch, ch), jnp.int32),
            [pltpu.VMEM((ch, d), jnp.float32)] * nbuf,
            pltpu.VMEM_SHARED((nseg, d), jnp.float32),
            [pltpu.SemaphoreType.DMA] * nbuf,
        ],
    )
    def sk(msg_hbm, idx_hbm, z_hbm, out_hbm, idx_v, bufs, acc_sh, rsems):
        cid = lax.axis_index("c")
        sid = lax.axis_index("s")
        wid = sid * _NC + cid
        base = wid * bpw
        pltpu.sync_copy(z_hbm, acc_sh.at[pl.ds(sid * nps, nps)])
        pltpu.sync_copy(idx_hbm.at[wid], idx_v)
        plsc.subcore_barrier()

        def step(i, bb):
            bi = bb
            bj = (bb - lag) % nbuf

            @pl.when(i < nch)
            def _():
                pltpu.async_copy(
                    msg_hbm.at[pl.ds(base + i * ch, ch)], bufs[bi], rsems[bi])

            j = i - lag

            @pl.when(jnp.logical_and(j >= 0, j < nch))
            def _():
                pltpu.make_async_copy(
                    msg_hbm.at[pl.ds(base, ch)], bufs[bj], rsems[bj]).wait()
                pltpu.sync_copy(bufs[bj], acc_sh.at[idx_v.at[j]], add=True)

        def body(g, carry):
            for bb in range(nbuf):
                step(g * nbuf + bb, bb)
            return carry

        lax.fori_loop(0, ngrp, body, 0)
        plsc.subcore_barrier()
        pltpu.sync_copy(acc_sh.at[pl.ds(sid * nps, nps)],
                        out_hbm.at[cid, pl.ds(sid * nps, nps)])

    return sk(msg, idx3, zblk)


def kernel(x, edge_index, edge_attr, node_time,
           nn_W1, nn_b1, nn_W2, nn_b2,
           en_W1, en_b1, en_W2, en_b2,
           msg_W, msg_b,
           g_W1, g_b1, g_W2, g_b2,
           cl_W, cl_b, ln_g, ln_b, out_W, out_b):
    n, nd = x.shape
    e, ed = edge_attr.shape
    hd = nn_W2.shape[1]

    def r2(v):
        return v.reshape(1, -1)

    gW1e = g_W1[:ed]
    gW1x = g_W1[ed:ed + nd]
    gW1t = g_W1[ed + nd:]

    row = edge_index[0]
    col = edge_index[1]
    ch = 80
    grp = _NW * ch
    nsl = 1
    ngroups = e // grp
    gsplit = [ngroups // nsl + (1 if i < ngroups % nsl else 0)
              for i in range(nsl)]

    bn = 5000
    T, skip = pl.pallas_call(
        _node_body,
        grid=(n // bn,),
        in_specs=[
            pl.BlockSpec((bn, nd), lambda i: (i, 0)),
            pl.BlockSpec((bn, 1), lambda i: (i, 0)),
            _full((nd, hd)), _full((1, hd)), _full((hd, hd)), _full((1, hd)),
            _full((nd, hd)), _full((1, hd)), _full((1, hd)),
            _full((nd, hd)), _full((1, hd)),
        ],
        out_specs=[
            pl.BlockSpec((bn, hd), lambda i: (i, 0)),
            pl.BlockSpec((bn, hd), lambda i: (i, 0)),
        ],
        out_shape=[
            jax.ShapeDtypeStruct((n, hd), jnp.int32),
            jax.ShapeDtypeStruct((n, hd), jnp.float32),
        ],
    )(x, node_time, nn_W1, r2(nn_b1), nn_W2, r2(nn_b2),
      gW1x, gW1t, r2(g_b1), cl_W, r2(cl_b))

    nps = -(-n // (_NS * 8)) * 8
    zblk = jnp.zeros((nps, hd), jnp.float32)
    be = 16000

    parts = []
    off = 0
    for s in range(nsl):
        nch = gsplit[s]
        eh = nch * grp
        col3 = lax.slice_in_dim(col, off, off + eh).reshape(_NW, nch, ch)
        row3 = lax.slice_in_dim(row, off, off + eh).reshape(_NW, nch, ch)
        G = _sc_gather(T, col3)
        blk0 = off // be
        off += eh
        msg = pl.pallas_call(
            _edge_body,
            grid=(eh // be,),
            in_specs=[
                pl.BlockSpec((be, ed), lambda i, b0=blk0: (i + b0, 0)),
                pl.BlockSpec((be, hd), lambda i: (i, 0)),
                _full((ed, hd)), _full((1, hd)), _full((hd, hd)), _full((1, hd)),
                _full((hd, hd)), _full((1, hd)),
                _full((ed, hd)), _full((hd, hd)), _full((1, hd)),
            ],
            out_specs=pl.BlockSpec((be, hd), lambda i: (i, 0)),
            out_shape=jax.ShapeDtypeStruct((eh, hd), jnp.float32),
        )(edge_attr, G, en_W1, r2(en_b1), en_W2, r2(en_b2),
          msg_W, r2(msg_b), gW1e, g_W2, r2(g_b2))
        parts.append(_sc_scatter(msg, row3, zblk))

    psum_specs = [pl.BlockSpec((bn, hd), lambda i: (i, 0))
                  for _ in range(2 * nsl)]
    out = pl.pallas_call(
        _final_body,
        grid=(n // bn,),
        in_specs=[
            pl.BlockSpec((bn, hd), lambda i: (i, 0)),
            *psum_specs,
            _full((1, hd)), _full((1, hd)),
            _full((hd, nd)), _full((1, nd)),
        ],
        out_specs=pl.BlockSpec((bn, nd), lambda i: (i, 0)),
        out_shape=jax.ShapeDtypeStruct((n, nd), jnp.float32),
    )(skip, *[p[c, :n] for p in parts for c in range(_NC)],
      r2(ln_g), r2(ln_b), out_W, r2(out_b))
    return out


# speedup vs baseline: 1.0111x; 1.0111x over previous
import functools

import jax
import jax.numpy as jnp
from jax import lax
from jax.experimental import pallas as pl
from jax.experimental.pallas import tpu as pltpu
from jax.experimental.pallas import tpu_sc as plsc

_NC = 2
_NS = 16
_NW = _NC * _NS


def _full(shape):
    return pl.BlockSpec(shape, lambda i: (0,) * len(shape))


def _round_bf16_bits(v):
    b = lax.bitcast_convert_type(v, jnp.uint32)
    return (b + jnp.uint32(0x7FFF) + ((b >> 16) & jnp.uint32(1))) >> 16


def _node_body(x_ref, nt_ref, nnW1, nnb1, nnW2, nnb2, gW1x, gW1t, gb1,
               T_ref):
    x = x_ref[...]
    h = jnp.maximum(x @ nnW1[...] + nnb1[...], 0.0) @ nnW2[...] + nnb2[...]
    g = x @ gW1x[...] + nt_ref[...] * gW1t[...] + gb1[...]
    w = _round_bf16_bits(h) | (_round_bf16_bits(g) << 16)
    T_ref[...] = lax.bitcast_convert_type(w, jnp.int32)


def _edge_body(ea_ref, G_ref, enW1, enb1, enW2, enb2, msgW, msgb,
               gW1e, gW2, gb2, out_ref):
    ea = ea_ref[...]
    he = jnp.maximum(ea @ enW1[...] + enb1[...], 0.0) @ enW2[...] + enb2[...]
    w = lax.bitcast_convert_type(G_ref[...], jnp.uint32)
    gh = lax.bitcast_convert_type(w << 16, jnp.float32)
    gg = lax.bitcast_convert_type(w & jnp.uint32(0xFFFF0000), jnp.float32)
    m = (he * gh) @ msgW[...] + msgb[...]
    gt = jnp.maximum(ea @ gW1e[...] + gg, 0.0) @ gW2[...] + gb2[...]
    out_ref[...] = m * jax.nn.sigmoid(gt)


def _final_body(x_ref, clW, clb, *rest):
    p_refs = rest[:-5]
    lng, lnb, outW, outb, out_ref = rest[-5:]
    o = x_ref[...] @ clW[...] + clb[...]
    for p in p_refs:
        o = o + p[...]
    mu = jnp.mean(o, axis=-1, keepdims=True)
    var = jnp.mean((o - mu) ** 2, axis=-1, keepdims=True)
    o = (o - mu) / jnp.sqrt(var + 1e-5) * lng[...] + lnb[...]
    out_ref[...] = jnp.maximum(o, 0.0) @ outW[...] + outb[...]


def _sc_gather(table, idx3):
    nt, d = table.shape
    nw, nch, ch = idx3.shape
    bpw = nch * ch
    b = nw * bpw
    nbuf = 3
    lag = 2
    ngrp = -(-(nch + lag) // nbuf)
    mesh = plsc.VectorSubcoreMesh(core_axis_name="c", subcore_axis_name="s")

    @functools.partial(
        pl.kernel,
        out_type=jax.ShapeDtypeStruct((b, d), table.dtype),
        mesh=mesh,
        scratch_types=[
            pltpu.VMEM((nch, ch), jnp.int32),
            [pltpu.VMEM((ch, d), table.dtype)] * nbuf,
            pltpu.VMEM_SHARED((nt, d), table.dtype),
            [pltpu.SemaphoreType.DMA] * nbuf,
            [pltpu.SemaphoreType.DMA] * nbuf,
        ],
    )
    def gk(table_hbm, idx_hbm, out_hbm, idx_v, bufs, tab_sh, gsems, wsems):
        cid = lax.axis_index("c")
        sid = lax.axis_index("s")
        wid = sid * _NC + cid
        base = wid * bpw
        @pl.when(sid == 0)
        def _():
            pltpu.sync_copy(table_hbm, tab_sh)

        pltpu.sync_copy(idx_hbm.at[wid], idx_v)
        plsc.subcore_barrier()

        def step(i, bb):
            bi = bb
            bj = (bb - lag) % nbuf

            @pl.when(jnp.logical_and(i >= nbuf, i < nch))
            def _():
                pltpu.make_async_copy(
                    bufs[bi], out_hbm.at[pl.ds(base, ch)], wsems[bi]).wait()

            @pl.when(i < nch)
            def _():
                pltpu.async_copy(tab_sh.at[idx_v.at[i]], bufs[bi], gsems[bi])

            j = i - lag

            @pl.when(jnp.logical_and(j >= 0, j < nch))
            def _():
                pltpu.make_async_copy(
                    tab_sh.at[idx_v.at[0]], bufs[bj], gsems[bj]).wait()
                pltpu.async_copy(
                    bufs[bj], out_hbm.at[pl.ds(base + j * ch, ch)], wsems[bj])

        def body(g, carry):
            for bb in range(nbuf):
                step(g * nbuf + bb, bb)
            return carry

        lax.fori_loop(0, ngrp, body, 0)
        for bb in range(min(nbuf, nch)):
            pltpu.make_async_copy(
                bufs[bb], out_hbm.at[pl.ds(base, ch)], wsems[bb]).wait()

    return gk(table, idx3)


def _sc_scatter(msg, idx3, zblk):
    b, d = msg.shape
    nw, nch, ch = idx3.shape
    bpw = nch * ch
    nps = zblk.shape[0]
    nseg = nps * _NS
    mesh = plsc.VectorSubcoreMesh(core_axis_name="c", subcore_axis_name="s")

    nbuf = 3
    lag = 2
    ngrp = -(-(nch + lag) // nbuf)

    @functools.partial(
        pl.kernel,
        out_type=jax.ShapeDtypeStruct((_NC, nseg, d), jnp.float32),
        mesh=mesh,
        scratch_types=[
            pltpu.VMEM((nch, ch), jnp.int32),
            [pltpu.VMEM((ch, d), jnp.float32)] * nbuf,
            pltpu.VMEM_SHARED((nseg, d), jnp.float32),
            [pltpu.SemaphoreType.DMA] * nbuf,
        ],
    )
    def sk(msg_hbm, idx_hbm, z_hbm, out_hbm, idx_v, bufs, acc_sh, rsems):
        cid = lax.axis_index("c")
        sid = lax.axis_index("s")
        wid = sid * _NC + cid
        base = wid * bpw
        pltpu.sync_copy(z_hbm, acc_sh.at[pl.ds(sid * nps, nps)])
        pltpu.sync_copy(idx_hbm.at[wid], idx_v)
        plsc.subcore_barrier()

        def step(i, bb):
            bi = bb
            bj = (bb - lag) % nbuf

            @pl.when(i < nch)
            def _():
                pltpu.async_copy(
                    msg_hbm.at[pl.ds(base + i * ch, ch)], bufs[bi], rsems[bi])

            j = i - lag

            @pl.when(jnp.logical_and(j >= 0, j < nch))
            def _():
                pltpu.make_async_copy(
                    msg_hbm.at[pl.ds(base, ch)], bufs[bj], rsems[bj]).wait()
                pltpu.sync_copy(bufs[bj], acc_sh.at[idx_v.at[j]], add=True)

        def body(g, carry):
            for bb in range(nbuf):
                step(g * nbuf + bb, bb)
            return carry

        lax.fori_loop(0, ngrp, body, 0)
        plsc.subcore_barrier()
        pltpu.sync_copy(acc_sh.at[pl.ds(sid * nps, nps)],
                        out_hbm.at[cid, pl.ds(sid * nps, nps)])

    return sk(msg, idx3, zblk)


def kernel(x, edge_index, edge_attr, node_time,
           nn_W1, nn_b1, nn_W2, nn_b2,
           en_W1, en_b1, en_W2, en_b2,
           msg_W, msg_b,
           g_W1, g_b1, g_W2, g_b2,
           cl_W, cl_b, ln_g, ln_b, out_W, out_b):
    n, nd = x.shape
    e, ed = edge_attr.shape
    hd = nn_W2.shape[1]

    def r2(v):
        return v.reshape(1, -1)

    gW1e = g_W1[:ed]
    gW1x = g_W1[ed:ed + nd]
    gW1t = g_W1[ed + nd:]

    row = edge_index[0]
    col = edge_index[1]
    ch = 80
    grp = _NW * ch
    nsl = 1
    ngroups = e // grp
    gsplit = [ngroups // nsl + (1 if i < ngroups % nsl else 0)
              for i in range(nsl)]

    bn = 5000
    T = pl.pallas_call(
        _node_body,
        grid=(n // bn,),
        in_specs=[
            pl.BlockSpec((bn, nd), lambda i: (i, 0)),
            pl.BlockSpec((bn, 1), lambda i: (i, 0)),
            _full((nd, hd)), _full((1, hd)), _full((hd, hd)), _full((1, hd)),
            _full((nd, hd)), _full((1, hd)), _full((1, hd)),
        ],
        out_specs=pl.BlockSpec((bn, hd), lambda i: (i, 0)),
        out_shape=jax.ShapeDtypeStruct((n, hd), jnp.int32),
    )(x, node_time, nn_W1, r2(nn_b1), nn_W2, r2(nn_b2),
      gW1x, gW1t, r2(g_b1))

    nps = -(-n // (_NS * 8)) * 8
    zblk = jnp.zeros((nps, hd), jnp.float32)
    be = 16000

    parts = []
    off = 0
    for s in range(nsl):
        nch = gsplit[s]
        eh = nch * grp
        col3 = lax.slice_in_dim(col, off, off + eh).reshape(_NW, nch, ch)
        row3 = lax.slice_in_dim(row, off, off + eh).reshape(_NW, nch, ch)
        G = _sc_gather(T, col3)
        blk0 = off // be
        off += eh
        msg = pl.pallas_call(
            _edge_body,
            grid=(eh // be,),
            in_specs=[
                pl.BlockSpec((be, ed), lambda i, b0=blk0: (i + b0, 0)),
                pl.BlockSpec((be, hd), lambda i: (i, 0)),
                _full((ed, hd)), _full((1, hd)), _full((hd, hd)), _full((1, hd)),
                _full((hd, hd)), _full((1, hd)),
                _full((ed, hd)), _full((hd, hd)), _full((1, hd)),
            ],
            out_specs=pl.BlockSpec((be, hd), lambda i: (i, 0)),
            out_shape=jax.ShapeDtypeStruct((eh, hd), jnp.float32),
        )(edge_attr, G, en_W1, r2(en_b1), en_W2, r2(en_b2),
          msg_W, r2(msg_b), gW1e, g_W2, r2(g_b2))
        parts.append(_sc_scatter(msg, row3, zblk))

    psum_specs = [pl.BlockSpec((bn, hd), lambda i: (i, 0))
                  for _ in range(2 * nsl)]
    out = pl.pallas_call(
        _final_body,
        grid=(n // bn,),
        in_specs=[
            pl.BlockSpec((bn, nd), lambda i: (i, 0)),
            _full((nd, hd)), _full((1, hd)),
            *psum_specs,
            _full((1, hd)), _full((1, hd)),
            _full((hd, nd)), _full((1, nd)),
        ],
        out_specs=pl.BlockSpec((bn, nd), lambda i: (i, 0)),
        out_shape=jax.ShapeDtypeStruct((n, nd), jnp.float32),
    )(x, cl_W, r2(cl_b), *[p[c, :n] for p in parts for c in range(_NC)],
      r2(ln_g), r2(ln_b), out_W, r2(out_b))
    return out
